# baseline (speedup 1.0000x reference)
import functools
import jax
import jax.numpy as jnp
from jax import lax
from jax.experimental import pallas as pl
from jax.experimental.pallas import tpu as pltpu
from jax.experimental.pallas import tpu_sc as plsc

NC = 2
NS = 16
LANES = 16
NW = NC * NS


def _emb_call(ids_flat, token_table, pos_table, *, seq, chunk):
    n = ids_flat.shape[0]
    d = token_table.shape[1]
    rows_per_w = n // NW
    n_chunks = rows_per_w // chunk
    d_vecs = d // LANES

    mesh = plsc.VectorSubcoreMesh(core_axis_name="c", subcore_axis_name="s")

    @functools.partial(
        pl.kernel,
        out_type=jax.ShapeDtypeStruct((n, d), jnp.float32),
        mesh=mesh,
        scratch_types=[
            pltpu.VMEM((chunk,), jnp.int32),
            pltpu.VMEM((chunk, d), jnp.float32),
            pltpu.VMEM((chunk, d), jnp.float32),
            pltpu.SemaphoreType.DMA,
        ],
    )
    def k(ids_hbm, tok_hbm, pos_hbm, out_hbm, idx_v, rows_v, pos_v, sem):
        wid = lax.axis_index("s") * NC + lax.axis_index("c")
        base = wid * rows_per_w
        pbase = lax.rem(base, seq)

        def chunk_body(ck, _):
            off = base + ck * chunk
            poff = pbase + ck * chunk
            pltpu.sync_copy(ids_hbm.at[pl.ds(off, chunk)], idx_v)
            gather = pltpu.async_copy(tok_hbm.at[idx_v], rows_v, sem)
            pltpu.sync_copy(pos_hbm.at[pl.ds(poff, chunk), :], pos_v)
            gather.wait()

            def row_body(i, _):
                for j in range(d_vecs):
                    sl = pl.ds(j * LANES, LANES)
                    plsc.addupdate(rows_v.at[i, sl], pos_v[i, sl])
                return 0

            lax.fori_loop(0, chunk, row_body, 0)
            pltpu.sync_copy(rows_v, out_hbm.at[pl.ds(off, chunk), :])
            return 0

        lax.fori_loop(0, n_chunks, chunk_body, 0)

    return k(ids_flat, token_table, pos_table)


def kernel(input_ids, token_table, pos_table):
    b, s = input_ids.shape
    d = token_table.shape[1]
    ids_flat = input_ids.reshape(-1).astype(jnp.int32)
    out = _emb_call(ids_flat, token_table, pos_table, seq=s, chunk=64)
    return out.reshape(b, s, d)


# speedup vs baseline: 1.1958x; 1.1958x over previous
import functools
import jax
import jax.numpy as jnp
from jax import lax
from jax.experimental import pallas as pl
from jax.experimental.pallas import tpu as pltpu
from jax.experimental.pallas import tpu_sc as plsc

NC = 2
NS = 16
LANES = 16
NW = NC * NS


def _emb_call(ids_flat, token_table, pos_table, *, batch, seq, chunk):
    d = token_table.shape[1]
    d_vecs = d // LANES
    ppw = seq // NW
    n_pchunks = ppw // chunk
    n_tasks = n_pchunks * batch

    mesh = plsc.VectorSubcoreMesh(core_axis_name="c", subcore_axis_name="s")

    @functools.partial(
        pl.kernel,
        out_type=jax.ShapeDtypeStruct((batch * seq, d), jnp.float32),
        mesh=mesh,
        scratch_types=[
            pltpu.VMEM((chunk,), jnp.int32),
            pltpu.VMEM((chunk,), jnp.int32),
            pltpu.VMEM((chunk, d), jnp.float32),
            pltpu.VMEM((chunk, d), jnp.float32),
            pltpu.VMEM((chunk, d), jnp.float32),
            pltpu.SemaphoreType.DMA,
            pltpu.SemaphoreType.DMA,
            pltpu.SemaphoreType.DMA,
            pltpu.SemaphoreType.DMA,
        ],
    )
    def k(ids_hbm, tok_hbm, pos_hbm, out_hbm,
          idx0, idx1, rows0, rows1, pos_v, gs0, gs1, os0, os1):
        wid = lax.axis_index("s") * NC + lax.axis_index("c")
        wpos = wid * ppw
        idx = [idx0, idx1]
        rows = [rows0, rows1]
        gsem = [gs0, gs1]
        osem = [os0, os1]
        store_h = [None, None]

        def row_off(t):
            p, b = t // batch, t % batch
            return b * seq + wpos + p * chunk

        def start_gather(t):
            buf = t & 1
            if store_h[buf] is not None:
                store_h[buf].wait()
            pltpu.sync_copy(ids_hbm.at[pl.ds(row_off(t), chunk)], idx[buf])
            return pltpu.async_copy(tok_hbm.at[idx[buf]], rows[buf], gsem[buf])

        def load_pos(p):
            pltpu.sync_copy(pos_hbm.at[pl.ds(wpos + p * chunk, chunk), :], pos_v)

        gather_h = start_gather(0)
        load_pos(0)
        for t in range(n_tasks):
            buf = t & 1
            nxt_h = start_gather(t + 1) if t + 1 < n_tasks else None
            gather_h.wait()
            gather_h = nxt_h
            if t % batch == 0 and t > 0:
                load_pos(t // batch)

            def row_body(i, _):
                for j in range(d_vecs):
                    sl = pl.ds(j * LANES, LANES)
                    plsc.addupdate(rows[buf].at[i, sl], pos_v[i, sl])
                return 0

            lax.fori_loop(0, chunk, row_body, 0)
            store_h[buf] = pltpu.async_copy(
                rows[buf], out_hbm.at[pl.ds(row_off(t), chunk), :], osem[buf])
        store_h[0].wait()
        store_h[1].wait()

    return k(ids_flat, token_table, pos_table)


def kernel(input_ids, token_table, pos_table):
    b, s = input_ids.shape
    d = token_table.shape[1]
    ids_flat = input_ids.reshape(-1).astype(jnp.int32)
    out = _emb_call(ids_flat, token_table, pos_table, batch=b, seq=s, chunk=32)
    return out.reshape(b, s, d)


# speedup vs baseline: 1.4306x; 1.1963x over previous
import functools
import jax
import jax.numpy as jnp
from jax import lax
from jax.experimental import pallas as pl
from jax.experimental.pallas import tpu as pltpu
from jax.experimental.pallas import tpu_sc as plsc

NC = 2
NS = 16
LANES = 16
NW = NC * NS
NBUF = 3


def _emb_call(ids_tasks, token_table, pos_table, *, batch, seq, chunk):
    d = token_table.shape[1]
    d_vecs = d // LANES
    ppw = seq // NW
    n_pchunks = ppw // chunk
    n_tasks = n_pchunks * batch

    mesh = plsc.VectorSubcoreMesh(core_axis_name="c", subcore_axis_name="s")

    @functools.partial(
        pl.kernel,
        out_type=jax.ShapeDtypeStruct((batch * seq, d), jnp.float32),
        mesh=mesh,
        scratch_types=[
            pltpu.VMEM((n_tasks, chunk), jnp.int32),
            [pltpu.VMEM((chunk, d), jnp.float32) for _ in range(NBUF)],
            [pltpu.VMEM((chunk, d), jnp.float32) for _ in range(2)],
            [pltpu.SemaphoreType.DMA for _ in range(NBUF)],
            [pltpu.SemaphoreType.DMA for _ in range(2)],
            [pltpu.SemaphoreType.DMA for _ in range(NBUF)],
        ],
    )
    def k(ids_hbm, tok_hbm, pos_hbm, out_hbm, idx_v, rows, pos, gsem, psem, osem):
        wid = lax.axis_index("s") * NC + lax.axis_index("c")
        wpos = wid * ppw
        store_h = [None] * NBUF
        pos_h = [None, None]
        gather_h = {}

        pltpu.sync_copy(ids_hbm.at[wid], idx_v)

        def row_off(t):
            p, b = t // batch, t % batch
            return b * seq + wpos + p * chunk

        def start_pos(p):
            pb = p & 1
            pos_h[pb] = pltpu.async_copy(
                pos_hbm.at[pl.ds(wpos + p * chunk, chunk), :], pos[pb], psem[pb])

        def start_gather(t):
            r = t % NBUF
            if store_h[r] is not None:
                store_h[r].wait()
            gather_h[t] = pltpu.async_copy(tok_hbm.at[idx_v.at[t]], rows[r], gsem[r])

        start_pos(0)
        for t in range(min(NBUF - 1, n_tasks)):
            start_gather(t)
        for t in range(n_tasks):
            r = t % NBUF
            p, b = t // batch, t % batch
            if t + NBUF - 1 < n_tasks:
                start_gather(t + NBUF - 1)
            if b == 0:
                pos_h[p & 1].wait()
                if p + 1 < n_pchunks:
                    start_pos(p + 1)
            gather_h.pop(t).wait()
            pbuf = pos[p & 1]

            def row_body(i, _):
                for j in range(d_vecs):
                    sl = pl.ds(j * LANES, LANES)
                    plsc.addupdate(rows[r].at[i, sl], pbuf[i, sl])
                return 0

            lax.fori_loop(0, chunk, row_body, 0)
            store_h[r] = pltpu.async_copy(
                rows[r], out_hbm.at[pl.ds(row_off(t), chunk), :], osem[r])
        for h in store_h:
            if h is not None:
                h.wait()

    return k(ids_tasks, token_table, pos_table)


def kernel(input_ids, token_table, pos_table):
    b, s = input_ids.shape
    d = token_table.shape[1]
    chunk = 32
    ppw = s // NW
    n_pchunks = ppw // chunk
    ids_tasks = (input_ids.astype(jnp.int32)
                 .reshape(b, NW, n_pchunks, chunk)
                 .transpose(1, 2, 0, 3)
                 .reshape(NW, n_pchunks * b, chunk))
    out = _emb_call(ids_tasks, token_table, pos_table, batch=b, seq=s, chunk=chunk)
    return out.reshape(b, s, d)
